# Initial kernel scaffold; baseline (speedup 1.0000x reference)
import jax
import jax.numpy as jnp
from jax.experimental import pallas as pl


def kernel(pv_y_osgb_fourier, pv_x_osgb_fourier, pv_system_row_number, pv_x_osgb, pv_time_utc_fourier, solar_azimuth, solar_elevation, embedding_table):
    raise NotImplementedError("write your pallas kernel here")


# speedup vs baseline: 1.5723x; 1.5723x over previous
import functools

import jax
import jax.numpy as jnp
from jax import lax
from jax.experimental import pallas as pl
from jax.experimental.pallas import tpu as pltpu
from jax.experimental.pallas import tpu_sc as plsc

NUM_GSPS = 360
B = 1024
N_PV = 512
F = 8
EMBED_DIM = 64
OUT_D = 2 * F + F + 2 + EMBED_DIM

NC = 2
NS = 16
NW = NC * NS
CHUNK = N_PV
CHUNKS_PER_W = (B * N_PV) // (NW * CHUNK)
G = 128
NG = CHUNK // G


def _sc_body(y_hbm, x_hbm, idx_hbm, packed_hbm, table_hbm, out_hbm,
             ybuf, xbuf, idxv, tbuf, embv, outv, sem):
    wid = lax.axis_index("s") * NC + lax.axis_index("c")
    lane = lax.iota(jnp.int32, 16)
    ymask = lane < 8

    def chunk_body(c, _):
        b = wid * CHUNKS_PER_W + c
        row_base = b * CHUNK

        pltpu.sync_copy(idx_hbm.at[pl.ds(b * NG, NG)], idxv)
        for j in range(NG):
            for k in range(G // 16):
                s = pl.ds(k * 16, 16)
                idxv[j, s] = idxv[j, s] + NUM_GSPS

        descs = [
            pltpu.async_copy(table_hbm.at[idxv.at[j]],
                             embv.at[pl.ds(j * G, G)], sem)
            for j in range(NG)
        ]

        pltpu.sync_copy(y_hbm.at[pl.ds(row_base * F, CHUNK * F)],
                        ybuf.at[pl.ds(0, CHUNK * F)])
        pltpu.sync_copy(x_hbm.at[pl.ds(row_base * F, CHUNK * F)],
                        xbuf.at[pl.ds(8, CHUNK * F)])
        pltpu.sync_copy(packed_hbm.at[b], tbuf)
        t16 = tbuf[...]

        for d in descs:
            d.wait()

        U = 4

        def row_body(i, _):
            for u in range(U):
                n = i * U + u
                ya = ybuf[pl.ds(n * F, 16)]
                xa = xbuf[pl.ds(n * F, 16)]
                outv[n, pl.ds(0, 16)] = jnp.where(ymask, ya, xa)
                outv[n, pl.ds(16, 16)] = t16
                outv[n, pl.ds(26, 16)] = embv[n, pl.ds(0, 16)]
                outv[n, pl.ds(42, 16)] = embv[n, pl.ds(16, 16)]
                outv[n, pl.ds(58, 16)] = embv[n, pl.ds(32, 16)]
                outv[n, pl.ds(74, 16)] = embv[n, pl.ds(48, 16)]
            return _

        lax.fori_loop(0, CHUNK // U, row_body, 0, unroll=False)

        pltpu.sync_copy(outv, out_hbm.at[pl.ds(row_base, CHUNK)])
        return _

    lax.fori_loop(0, CHUNKS_PER_W, chunk_body, 0, unroll=False)


@functools.partial(jax.jit, static_argnames=("interpret",))
def _pv_query(y_flat, x_flat, idx2d, packed, table, interpret=False):
    mesh = plsc.VectorSubcoreMesh(core_axis_name="c", subcore_axis_name="s",
                                  num_cores=NC, num_subcores=NS)
    fn = pl.kernel(
        _sc_body,
        out_type=jax.ShapeDtypeStruct((B * N_PV, OUT_D), jnp.float32),
        mesh=mesh,
        scratch_types=[
            pltpu.VMEM((CHUNK * F + 16,), jnp.float32),
            pltpu.VMEM((CHUNK * F + 16,), jnp.float32),
            pltpu.VMEM((NG, G), jnp.int32),
            pltpu.VMEM((16,), jnp.float32),
            pltpu.VMEM((CHUNK, EMBED_DIM), jnp.float32),
            pltpu.VMEM((CHUNK, OUT_D), jnp.float32),
            pltpu.SemaphoreType.DMA,
        ],
        compiler_params=pltpu.CompilerParams(use_tc_tiling_on_sc=False),
        interpret=interpret,
    )
    return fn(y_flat, x_flat, idx2d, packed, table)


def kernel(pv_y_osgb_fourier, pv_x_osgb_fourier, pv_system_row_number,
           pv_x_osgb, pv_time_utc_fourier, solar_azimuth, solar_elevation,
           embedding_table):
    del pv_x_osgb
    y_flat = pv_y_osgb_fourier.reshape(B * N_PV * F)
    x_flat = pv_x_osgb_fourier.reshape(B * N_PV * F)
    idx2d = pv_system_row_number.astype(jnp.int32).reshape((B * N_PV) // G, G)
    packed = jnp.concatenate(
        [pv_time_utc_fourier,
         solar_azimuth[:, None],
         solar_elevation[:, None],
         jnp.zeros((B, 6), jnp.float32)], axis=1)
    out = _pv_query(y_flat, x_flat, idx2d, packed, embedding_table)
    return out.reshape(B, N_PV, OUT_D)


# speedup vs baseline: 1.5736x; 1.0008x over previous
import functools

import jax
import jax.numpy as jnp
from jax import lax
from jax.experimental import pallas as pl
from jax.experimental.pallas import tpu as pltpu
from jax.experimental.pallas import tpu_sc as plsc

NUM_GSPS = 360
B = 1024
N_PV = 512
F = 8
EMBED_DIM = 64
OUT_D = 2 * F + F + 2 + EMBED_DIM

NC = 2
NS = 16
NW = NC * NS
ROWS_W = (B * N_PV) // NW
G = 128
C = G
NCH = ROWS_W // C
D = 4
U = 8


def _sc_body(y_hbm, x_hbm, idx_hbm, packed_hbm, table_hbm, out_hbm,
             idxv, tbuf, embv, outv, gsem, isem, osem):
    wid = lax.axis_index("s") * NC + lax.axis_index("c")
    wbase = wid * ROWS_W

    def prefetch(i, p, in_loop):
        row_base = wbase + i * C
        b = wid * (ROWS_W // N_PV) + lax.shift_right_logical(i, 2)

        if in_loop:
            @pl.when(i >= D)
            def _():
                pltpu.make_async_copy(
                    outv.at[p], out_hbm.at[pl.ds(row_base, C)],
                    osem.at[p]).wait()

        pltpu.sync_copy(idx_hbm.at[wid * NCH + i], idxv.at[p])
        for k in range(G // 16):
            s = pl.ds(k * 16, 16)
            idxv[p, s] = idxv[p, s] + NUM_GSPS
        pltpu.async_copy(table_hbm.at[idxv.at[p]], embv.at[p], gsem.at[p])
        pltpu.async_copy(y_hbm.at[pl.ds(row_base, C)],
                         outv.at[p, :, pl.ds(0, F)], isem.at[p])
        pltpu.async_copy(x_hbm.at[pl.ds(row_base, C)],
                         outv.at[p, :, pl.ds(F, F)], isem.at[p])
        pltpu.async_copy(packed_hbm.at[b], tbuf.at[p], isem.at[p])

    def compute(c, p):
        row_base = wbase + c * C
        b = wid * (ROWS_W // N_PV) + lax.shift_right_logical(c, 2)
        pltpu.make_async_copy(y_hbm.at[pl.ds(row_base, C)],
                              outv.at[p, :, pl.ds(0, F)], isem.at[p]).wait()
        pltpu.make_async_copy(x_hbm.at[pl.ds(row_base, C)],
                              outv.at[p, :, pl.ds(F, F)], isem.at[p]).wait()
        pltpu.make_async_copy(packed_hbm.at[b], tbuf.at[p], isem.at[p]).wait()
        pltpu.make_async_copy(table_hbm.at[idxv.at[p]], embv.at[p],
                              gsem.at[p]).wait()
        t16 = tbuf[p, pl.ds(0, 16)]

        def row_body(r, _):
            for u in range(U):
                n = r * U + u
                outv[p, n, pl.ds(16, 16)] = t16
                outv[p, n, pl.ds(26, 16)] = embv[p, n, pl.ds(0, 16)]
                outv[p, n, pl.ds(42, 16)] = embv[p, n, pl.ds(16, 16)]
                outv[p, n, pl.ds(58, 16)] = embv[p, n, pl.ds(32, 16)]
                outv[p, n, pl.ds(74, 16)] = embv[p, n, pl.ds(48, 16)]
            return _

        lax.fori_loop(0, C // U, row_body, 0, unroll=False)
        pltpu.async_copy(outv.at[p], out_hbm.at[pl.ds(row_base, C)], osem.at[p])

    prefetch(0, 0, in_loop=False)

    def step(h, carry):
        i0 = D * h
        for s in range(D):
            i = i0 + s

            @pl.when(i + 1 < NCH)
            def _pf(i=i, s=s):
                prefetch(i + 1, (s + 1) % D, in_loop=True)

            compute(i, s)
        return carry

    lax.fori_loop(0, NCH // D, step, 0, unroll=False)

    for p in range(D):
        pltpu.make_async_copy(outv.at[p], out_hbm.at[pl.ds(wbase, C)],
                              osem.at[p]).wait()


@functools.partial(jax.jit, static_argnames=("interpret",))
def _pv_query(y2d, x2d, idx2d, packed, table, interpret=False):
    mesh = plsc.VectorSubcoreMesh(core_axis_name="c", subcore_axis_name="s",
                                  num_cores=NC, num_subcores=NS)
    fn = pl.kernel(
        _sc_body,
        out_type=jax.ShapeDtypeStruct((B * N_PV, OUT_D), jnp.float32),
        mesh=mesh,
        scratch_types=[
            pltpu.VMEM((D, G), jnp.int32),
            pltpu.VMEM((D, 16), jnp.float32),
            pltpu.VMEM((D, C, EMBED_DIM), jnp.float32),
            pltpu.VMEM((D, C, OUT_D), jnp.float32),
            pltpu.SemaphoreType.DMA((D,)),
            pltpu.SemaphoreType.DMA((D,)),
            pltpu.SemaphoreType.DMA((D,)),
        ],
        compiler_params=pltpu.CompilerParams(use_tc_tiling_on_sc=False),
        interpret=interpret,
    )
    return fn(y2d, x2d, idx2d, packed, table)


def kernel(pv_y_osgb_fourier, pv_x_osgb_fourier, pv_system_row_number,
           pv_x_osgb, pv_time_utc_fourier, solar_azimuth, solar_elevation,
           embedding_table):
    del pv_x_osgb
    y2d = pv_y_osgb_fourier.reshape(B * N_PV, F)
    x2d = pv_x_osgb_fourier.reshape(B * N_PV, F)
    idx2d = pv_system_row_number.astype(jnp.int32).reshape((B * N_PV) // G, G)
    packed = jnp.concatenate(
        [pv_time_utc_fourier,
         solar_azimuth[:, None],
         solar_elevation[:, None],
         jnp.zeros((B, 6), jnp.float32)], axis=1)
    out = _pv_query(y2d, x2d, idx2d, packed, embedding_table)
    return out.reshape(B, N_PV, OUT_D)
